# baseline (speedup 1.0000x reference)
import functools

import jax
import jax.numpy as jnp
from jax import lax
from jax.experimental import pallas as pl
from jax.experimental.pallas import tpu as pltpu
from jax.experimental.pallas import tpu_sc as plsc

BATCH = 16384
FIELDS = 26
EMBED_DIM = 32
VOCAB = 1000000
PADDED_DIM = 128

NUM_CORES = 2
NUM_SUBCORES = 16
NW = NUM_CORES * NUM_SUBCORES
BLK = 128
NBB = BATCH // BLK
BB_PER_W = NBB // NW
IDS_PER_W = BB_PER_W * BLK
TILES_PER_W = FIELDS * BB_PER_W

_mesh = plsc.VectorSubcoreMesh(
    core_axis_name="c",
    subcore_axis_name="s",
    num_cores=NUM_CORES,
    num_subcores=NUM_SUBCORES,
)


@functools.partial(
    pl.kernel,
    mesh=_mesh,
    out_type=jax.ShapeDtypeStruct((FIELDS, EMBED_DIM, BATCH), jnp.float32),
    scratch_types=[
        pltpu.VMEM((FIELDS * IDS_PER_W,), jnp.int32),
        pltpu.VMEM((BLK, PADDED_DIM), jnp.float32),
        pltpu.VMEM((EMBED_DIM, BLK), jnp.float32),
        pltpu.SemaphoreType.DMA,
    ],
    compiler_params=pltpu.CompilerParams(needs_layout_passes=False),
)
def _gather_kernel(tpad_hbm, idsf_hbm, out_hbm, idx_v, gbuf, trans, gsem):
    wid = lax.axis_index("s") * NUM_CORES + lax.axis_index("c")
    for f in range(FIELDS):
        pltpu.sync_copy(
            idsf_hbm.at[pl.ds(f * BATCH + wid * IDS_PER_W, IDS_PER_W)],
            idx_v.at[pl.ds(f * IDS_PER_W, IDS_PER_W)],
        )

    bidx = [lax.iota(jnp.int32, 16) + g * 16 for g in range(8)]

    def tile_body(t, carry):
        f = t // BB_PER_W
        bl = t % BB_PER_W
        bb = wid * BB_PER_W + bl
        pltpu.async_copy(
            tpad_hbm.at[idx_v.at[pl.ds(f * IDS_PER_W + bl * BLK, BLK)]],
            gbuf,
            gsem,
        ).wait()
        for e in range(EMBED_DIM):
            eidx = jnp.full((16,), e, jnp.int32)
            for g in range(8):
                trans[e, pl.ds(g * 16, 16)] = plsc.load_gather(
                    gbuf, [bidx[g], eidx]
                )
        pltpu.sync_copy(trans, out_hbm.at[f, :, pl.ds(bb * BLK, BLK)])
        return carry

    lax.fori_loop(0, TILES_PER_W, tile_body, 0)


def kernel(ids, table):
    tpad = jnp.pad(table, ((0, 0), (0, PADDED_DIM - EMBED_DIM)))
    idsf = ids.T.reshape(-1)
    out3 = _gather_kernel(tpad, idsf)
    return out3.transpose(2, 0, 1)


# speedup vs baseline: 1.0113x; 1.0113x over previous
import functools

import jax
import jax.numpy as jnp
from jax import lax
from jax.experimental import pallas as pl
from jax.experimental.pallas import tpu as pltpu
from jax.experimental.pallas import tpu_sc as plsc

BATCH = 16384
FIELDS = 26
EMBED_DIM = 32
VOCAB = 1000000
PADDED_DIM = 128

NUM_CORES = 2
NUM_SUBCORES = 16
NW = NUM_CORES * NUM_SUBCORES
BLK = 128
NBB = BATCH // BLK
BB_PER_W = NBB // NW
IDS_PER_W = BB_PER_W * BLK
TILES_PER_W = FIELDS * BB_PER_W
NBUF = 4

_mesh = plsc.VectorSubcoreMesh(
    core_axis_name="c",
    subcore_axis_name="s",
    num_cores=NUM_CORES,
    num_subcores=NUM_SUBCORES,
)


@functools.partial(
    pl.kernel,
    mesh=_mesh,
    out_type=jax.ShapeDtypeStruct((FIELDS, EMBED_DIM, BATCH), jnp.float32),
    scratch_types=[
        pltpu.VMEM((FIELDS, IDS_PER_W), jnp.int32),
        [pltpu.VMEM((BLK, PADDED_DIM), jnp.float32) for _ in range(NBUF)],
        pltpu.VMEM((EMBED_DIM, BLK), jnp.float32),
        [pltpu.SemaphoreType.DMA for _ in range(NBUF)],
    ],
    compiler_params=pltpu.CompilerParams(needs_layout_passes=False),
)
def _gather_kernel(tpad_hbm, idsT_hbm, out_hbm, idsv, gbufs, trans, gsems):
    wid = lax.axis_index("s") * NUM_CORES + lax.axis_index("c")
    pltpu.sync_copy(idsT_hbm.at[:, pl.ds(wid * IDS_PER_W, IDS_PER_W)], idsv)

    bidx = [lax.iota(jnp.int32, 16) + g * 16 for g in range(8)]
    eidx = [jnp.full((16,), e, jnp.int32) for e in range(EMBED_DIM)]

    def start_gather(t, b):
        f = t // BB_PER_W
        bl = t % BB_PER_W
        return pltpu.async_copy(
            tpad_hbm.at[idsv.at[f, pl.ds(bl * BLK, BLK)]], gbufs[b], gsems[b]
        )

    def body(t0, carry):
        gathers = [start_gather(t0 + b, b) for b in range(NBUF)]
        for b in range(NBUF):
            t = t0 + b
            f = t // BB_PER_W
            bb = wid * BB_PER_W + t % BB_PER_W
            gathers[b].wait()
            for e in range(EMBED_DIM):
                for g in range(8):
                    trans[e, pl.ds(g * 16, 16)] = plsc.load_gather(
                        gbufs[b], [bidx[g], eidx[e]]
                    )
            pltpu.sync_copy(trans, out_hbm.at[f, :, pl.ds(bb * BLK, BLK)])
        return carry

    lax.fori_loop(0, TILES_PER_W // NBUF, lambda i, c: body(i * NBUF, c), 0)


def kernel(ids, table):
    tpad = jnp.pad(table, ((0, 0), (0, PADDED_DIM - EMBED_DIM)))
    out3 = _gather_kernel(tpad, ids.T)
    return out3.transpose(2, 0, 1)


# speedup vs baseline: 1.1258x; 1.1132x over previous
import functools

import jax
import jax.numpy as jnp
from jax import lax
from jax.experimental import pallas as pl
from jax.experimental.pallas import tpu as pltpu
from jax.experimental.pallas import tpu_sc as plsc

BATCH = 16384
FIELDS = 26
EMBED_DIM = 32
VOCAB = 1000000
PACK = 4
PACKED_ROWS = VOCAB // PACK

NUM_CORES = 2
NUM_SUBCORES = 16
NW = NUM_CORES * NUM_SUBCORES
BLK = 128
NBB = BATCH // BLK
BB_PER_W = NBB // NW
IDS_PER_W = BB_PER_W * BLK
TILES_PER_W = FIELDS * BB_PER_W
NBUF = 4

_mesh = plsc.VectorSubcoreMesh(
    core_axis_name="c",
    subcore_axis_name="s",
    num_cores=NUM_CORES,
    num_subcores=NUM_SUBCORES,
)


@functools.partial(
    pl.kernel,
    mesh=_mesh,
    out_type=jax.ShapeDtypeStruct((FIELDS, EMBED_DIM, BATCH), jnp.float32),
    scratch_types=[
        pltpu.VMEM((FIELDS, IDS_PER_W), jnp.int32),
        [pltpu.VMEM((BLK,), jnp.int32) for _ in range(NBUF)],
        [pltpu.VMEM((BLK, 128), jnp.float32) for _ in range(NBUF)],
        pltpu.VMEM((EMBED_DIM, BLK), jnp.float32),
        [pltpu.SemaphoreType.DMA for _ in range(NBUF)],
    ],
    compiler_params=pltpu.CompilerParams(needs_layout_passes=False),
)
def _gather_kernel(tq_hbm, idsT_hbm, out_hbm, idsv, idxqs, gbufs, trans, gsems):
    wid = lax.axis_index("s") * NUM_CORES + lax.axis_index("c")
    pltpu.sync_copy(idsT_hbm.at[:, pl.ds(wid * IDS_PER_W, IDS_PER_W)], idsv)

    bidx = [lax.iota(jnp.int32, 16) + g * 16 for g in range(8)]

    def load_chunks(t):
        f = t // BB_PER_W
        bl = t % BB_PER_W
        return [
            idsv[f, pl.ds(bl * BLK + g * 16, 16)] for g in range(8)
        ]

    def body(t0, carry):
        chunks = []
        gathers = []
        for b in range(NBUF):
            ch = load_chunks(t0 + b)
            chunks.append(ch)
            for g in range(8):
                idxqs[b][pl.ds(g * 16, 16)] = lax.shift_right_logical(ch[g], 2)
            gathers.append(
                pltpu.async_copy(tq_hbm.at[idxqs[b]], gbufs[b], gsems[b])
            )
        for b in range(NBUF):
            t = t0 + b
            f = t // BB_PER_W
            bb = wid * BB_PER_W + t % BB_PER_W
            emod = [
                lax.shift_left(jnp.bitwise_and(chunks[b][g], 3), 5)
                for g in range(8)
            ]
            gathers[b].wait()
            for e in range(EMBED_DIM):
                vals = [
                    plsc.load_gather(gbufs[b], [bidx[g], emod[g] + e])
                    for g in range(8)
                ]
                for g in range(8):
                    trans[e, pl.ds(g * 16, 16)] = vals[g]
            pltpu.sync_copy(trans, out_hbm.at[f, :, pl.ds(bb * BLK, BLK)])
        return carry

    lax.fori_loop(0, TILES_PER_W // NBUF, lambda i, c: body(i * NBUF, c), 0)


def kernel(ids, table):
    tq = table.reshape(PACKED_ROWS, PACK * EMBED_DIM)
    out3 = _gather_kernel(tq, ids.T)
    return out3.transpose(2, 0, 1)


# speedup vs baseline: 1.1731x; 1.0420x over previous
import functools

import jax
import jax.numpy as jnp
from jax import lax
from jax.experimental import pallas as pl
from jax.experimental.pallas import tpu as pltpu
from jax.experimental.pallas import tpu_sc as plsc

BATCH = 16384
FIELDS = 26
EMBED_DIM = 32
VOCAB = 1000000

NUM_CORES = 2
NUM_SUBCORES = 16
NW = NUM_CORES * NUM_SUBCORES
BLK = 128
NBB = BATCH // BLK
BB_PER_W = NBB // NW
IDS_PER_W = BB_PER_W * BLK
TILES_PER_W = FIELDS * BB_PER_W
NBUF = 4

_mesh = plsc.VectorSubcoreMesh(
    core_axis_name="c",
    subcore_axis_name="s",
    num_cores=NUM_CORES,
    num_subcores=NUM_SUBCORES,
)


@functools.partial(
    pl.kernel,
    mesh=_mesh,
    out_type=jax.ShapeDtypeStruct((FIELDS, 4, NBB, 8, BLK), jnp.float32),
    scratch_types=[
        pltpu.VMEM((FIELDS * IDS_PER_W,), jnp.int32),
        [pltpu.VMEM((BLK, EMBED_DIM), jnp.float32) for _ in range(NBUF)],
        pltpu.VMEM((EMBED_DIM, BLK), jnp.float32),
        [pltpu.SemaphoreType.DMA for _ in range(NBUF)],
    ],
    compiler_params=pltpu.CompilerParams(
        use_tc_tiling_on_sc=False, needs_layout_passes=False
    ),
)
def _gather_kernel(t_hbm, idsf_hbm, out_hbm, idsv, gbufs, trans, gsems):
    wid = lax.axis_index("s") * NUM_CORES + lax.axis_index("c")
    for f in range(FIELDS):
        pltpu.sync_copy(
            idsf_hbm.at[pl.ds(f * BATCH + wid * IDS_PER_W, IDS_PER_W)],
            idsv.at[pl.ds(f * IDS_PER_W, IDS_PER_W)],
        )

    bidx = [lax.iota(jnp.int32, 16) + g * 16 for g in range(8)]
    eidx = [jnp.full((16,), e, jnp.int32) for e in range(EMBED_DIM)]

    def body(t0, carry):
        gathers = []
        for b in range(NBUF):
            t = t0 + b
            f = t // BB_PER_W
            bl = t % BB_PER_W
            gathers.append(
                pltpu.async_copy(
                    t_hbm.at[idsv.at[pl.ds(f * IDS_PER_W + bl * BLK, BLK)]],
                    gbufs[b],
                    gsems[b],
                )
            )
        for b in range(NBUF):
            t = t0 + b
            f = t // BB_PER_W
            bb = wid * BB_PER_W + t % BB_PER_W
            gathers[b].wait()
            for e in range(EMBED_DIM):
                vals = [
                    plsc.load_gather(gbufs[b], [bidx[g], eidx[e]])
                    for g in range(8)
                ]
                for g in range(8):
                    trans[e, pl.ds(g * 16, 16)] = vals[g]
            for eb in range(4):
                pltpu.sync_copy(
                    trans.at[pl.ds(eb * 8, 8), :], out_hbm.at[f, eb, bb]
                )
        return carry

    lax.fori_loop(0, TILES_PER_W // NBUF, lambda i, c: body(i * NBUF, c), 0)


def kernel(ids, table):
    idsf = ids.T.reshape(-1)
    out5 = _gather_kernel(table, idsf)
    return out5.transpose(2, 4, 0, 1, 3).reshape(BATCH, FIELDS, EMBED_DIM)


# speedup vs baseline: 1.3258x; 1.1301x over previous
import functools

import jax
import jax.numpy as jnp
from jax import lax
from jax.experimental import pallas as pl
from jax.experimental.pallas import tpu as pltpu
from jax.experimental.pallas import tpu_sc as plsc

BATCH = 16384
FIELDS = 26
EMBED_DIM = 32
VOCAB = 1000000

NUM_CORES = 2
NUM_SUBCORES = 16
NW = NUM_CORES * NUM_SUBCORES
BLK = 128
NBB = BATCH // BLK
BB_PER_W = NBB // NW
IDS_PER_W = BB_PER_W * BLK
TILES_PER_W = FIELDS * BB_PER_W
NBUF = 4

_mesh = plsc.VectorSubcoreMesh(
    core_axis_name="c",
    subcore_axis_name="s",
    num_cores=NUM_CORES,
    num_subcores=NUM_SUBCORES,
)


@functools.partial(
    pl.kernel,
    mesh=_mesh,
    out_type=jax.ShapeDtypeStruct((FIELDS, 4, NBB, 8, BLK), jnp.float32),
    scratch_types=[
        pltpu.VMEM((FIELDS * IDS_PER_W,), jnp.int32),
        [pltpu.VMEM((BLK, EMBED_DIM), jnp.float32) for _ in range(NBUF)],
        pltpu.VMEM((4, 8, BLK), jnp.float32),
        [pltpu.SemaphoreType.DMA for _ in range(NBUF)],
    ],
    compiler_params=pltpu.CompilerParams(
        use_tc_tiling_on_sc=False, needs_layout_passes=False
    ),
)
def _gather_kernel(t_hbm, idsf_hbm, out_hbm, idsv, gbufs, trans, gsems):
    wid = lax.axis_index("s") * NUM_CORES + lax.axis_index("c")
    for f in range(FIELDS):
        pltpu.sync_copy(
            idsf_hbm.at[pl.ds(f * BATCH + wid * IDS_PER_W, IDS_PER_W)],
            idsv.at[pl.ds(f * IDS_PER_W, IDS_PER_W)],
        )

    bidx = [lax.iota(jnp.int32, 16) + g * 16 for g in range(8)]
    eidx = [jnp.full((16,), e, jnp.int32) for e in range(EMBED_DIM)]

    def body(t0, carry):
        gathers = []
        for b in range(NBUF):
            t = t0 + b
            f = t // BB_PER_W
            bl = t % BB_PER_W
            gathers.append(
                pltpu.async_copy(
                    t_hbm.at[idsv.at[pl.ds(f * IDS_PER_W + bl * BLK, BLK)]],
                    gbufs[b],
                    gsems[b],
                )
            )
        for b in range(NBUF):
            t = t0 + b
            f = t // BB_PER_W
            bb = wid * BB_PER_W + t % BB_PER_W
            gathers[b].wait()
            gbuf = gbufs[b]

            @plsc.parallel_loop(0, EMBED_DIM, step=1, unroll=8)
            def _transpose(e):
                ev = jnp.full((16,), 1, jnp.int32) * e
                eb_i = lax.shift_right_logical(e, 3)
                es_i = jnp.bitwise_and(e, 7)
                for g in range(8):
                    trans[eb_i, es_i, pl.ds(g * 16, 16)] = plsc.load_gather(
                        gbuf, [bidx[g], ev]
                    )

            pltpu.sync_copy(trans, out_hbm.at[f, :, bb])
        return carry

    lax.fori_loop(0, TILES_PER_W // NBUF, lambda i, c: body(i * NBUF, c), 0)


def kernel(ids, table):
    idsf = ids.T.reshape(-1)
    out5 = _gather_kernel(table, idsf)
    return out5.transpose(2, 4, 0, 1, 3).reshape(BATCH, FIELDS, EMBED_DIM)
